# Initial kernel scaffold; baseline (speedup 1.0000x reference)
import jax
import jax.numpy as jnp
from jax.experimental import pallas as pl


def kernel(features, edge_index, W0, b0, W1, b1, W2, b2, pool_weight):
    raise NotImplementedError("write your pallas kernel here")


# speedup vs baseline: 4.9768x; 4.9768x over previous
import functools

import jax
import jax.numpy as jnp
from jax import lax
from jax.experimental import pallas as pl
from jax.experimental.pallas import tpu as pltpu
from jax.experimental.pallas import tpu_sc as plsc

N = 10000
D = 128
E = 320000
NC = 2
NS = 16
CH = 128
E_PAD = 323584
N_PAD = 10240
RPT = N_PAD // NS


def _sc_mesh():
    return plsc.VectorSubcoreMesh(core_axis_name="c", subcore_axis_name="s")


def _sc_degree(idx2, zeros128, ones128):
    @functools.partial(
        pl.kernel,
        mesh=_sc_mesh(),
        out_type=jax.ShapeDtypeStruct((2 * N_PAD, D), jnp.float32),
        scratch_types=[
            pltpu.VMEM((CH,), jnp.int32),
            pltpu.VMEM((CH, D), jnp.float32),
            pltpu.VMEM_SHARED((N_PAD, D), jnp.float32),
        ],
    )
    def k(idx_hbm, z_hbm, o_hbm, deg_hbm, idx_v, ones_v, acc):
        c = lax.axis_index("c")
        s = lax.axis_index("s")
        pltpu.sync_copy(z_hbm, acc.at[pl.ds(s * RPT, RPT)])
        pltpu.sync_copy(o_hbm, ones_v)
        plsc.subcore_barrier()
        per_tile = E_PAD // NS

        def body(g, carry):
            off = c * E_PAD + s * per_tile + g * CH
            pltpu.sync_copy(idx_hbm.at[pl.ds(off, CH)], idx_v)
            pltpu.sync_copy(ones_v, acc.at[idx_v], add=True)
            return carry

        lax.fori_loop(0, per_tile // CH, body, 0)
        plsc.subcore_barrier()
        pltpu.sync_copy(
            acc.at[pl.ds(s * RPT, RPT)],
            deg_hbm.at[pl.ds(c * N_PAD + s * RPT, RPT)],
        )

    return k(idx2, zeros128, ones128)


def _sc_scatter(hn, src_m, dst_m, zeros128):
    @functools.partial(
        pl.kernel,
        mesh=_sc_mesh(),
        out_type=jax.ShapeDtypeStruct((2 * N_PAD, D), jnp.float32),
        scratch_types=[
            pltpu.VMEM((CH,), jnp.int32),
            pltpu.VMEM((CH,), jnp.int32),
            pltpu.VMEM((CH, D), jnp.float32),
            pltpu.SemaphoreType.DMA,
            pltpu.VMEM_SHARED((N_PAD, D), jnp.float32),
        ],
    )
    def k(hn_hbm, src_hbm, dst_hbm, z_hbm, out_hbm, sidx, didx, rows, sem, acc):
        c = lax.axis_index("c")
        s = lax.axis_index("s")
        wid = s * NC + c
        pltpu.sync_copy(z_hbm, acc.at[pl.ds(s * RPT, RPT)])
        plsc.subcore_barrier()
        per_tile = E_PAD // (NC * NS)

        def body(g, carry):
            off = wid * per_tile + g * CH
            pltpu.sync_copy(src_hbm.at[pl.ds(off, CH)], sidx)
            pltpu.sync_copy(dst_hbm.at[pl.ds(off, CH)], didx)
            pltpu.async_copy(hn_hbm.at[sidx], rows, sem).wait()
            pltpu.sync_copy(rows, acc.at[didx], add=True)
            return carry

        lax.fori_loop(0, per_tile // CH, body, 0)
        plsc.subcore_barrier()
        pltpu.sync_copy(
            acc.at[pl.ds(s * RPT, RPT)],
            out_hbm.at[pl.ds(c * N_PAD + s * RPT, RPT)],
        )

    return k(hn, src_m, dst_m, zeros128)


def _first_body(x_ref, w_ref, deg_ref, o_ref):
    nsrc = lax.rsqrt(jnp.maximum(deg_ref[:, 0:1], 1.0))
    o_ref[...] = (
        jnp.dot(x_ref[...], w_ref[...], preferred_element_type=jnp.float32)
        * nsrc
    )


def _tc_first(x, w, deg_out):
    return pl.pallas_call(
        _first_body,
        out_shape=jax.ShapeDtypeStruct((N, D), jnp.float32),
    )(x, w, deg_out)


def _mid_body(p0_ref, p1_ref, din_ref, dout_ref, b_ref, w_ref, o_ref):
    ndst = lax.rsqrt(jnp.maximum(din_ref[:, 0:1], 1.0))
    t = jnp.maximum((p0_ref[...] + p1_ref[...]) * ndst + b_ref[...], 0.0)
    nsrc = lax.rsqrt(jnp.maximum(dout_ref[:, 0:1], 1.0))
    o_ref[...] = (
        jnp.dot(t, w_ref[...], preferred_element_type=jnp.float32) * nsrc
    )


def _tc_mid(p0, p1, deg_in, deg_out, b, w):
    return pl.pallas_call(
        _mid_body,
        out_shape=jax.ShapeDtypeStruct((N, D), jnp.float32),
    )(p0, p1, deg_in, deg_out, b, w)


def _final_body(p0_ref, p1_ref, din_ref, b_ref, wm_ref, wa_ref, o_ref):
    ndst = lax.rsqrt(jnp.maximum(din_ref[:, 0:1], 1.0))
    t = jnp.maximum((p0_ref[...] + p1_ref[...]) * ndst + b_ref[...], 0.0)
    mx = jnp.max(t, axis=0, keepdims=True)
    sm = jnp.sum(t, axis=0, keepdims=True)
    o_ref[...] = wm_ref[...] * mx + (wa_ref[...] / N) * sm


def _tc_final(p0, p1, deg_in, b, wm, wa):
    return pl.pallas_call(
        _final_body,
        out_shape=jax.ShapeDtypeStruct((1, D), jnp.float32),
    )(p0, p1, deg_in, b, wm, wa)


def kernel(features, edge_index, W0, b0, W1, b1, W2, b2, pool_weight):
    src = edge_index[0]
    dst = edge_index[1]
    npad = E_PAD - E
    pad_sink = jnp.full((npad,), N, dtype=jnp.int32)
    pad_zero = jnp.zeros((npad,), dtype=jnp.int32)
    src_m = jnp.concatenate([src, pad_zero])
    dst_m = jnp.concatenate([dst, pad_sink])
    idx2 = jnp.concatenate([src, pad_sink, dst, pad_sink])
    ones128 = jnp.ones((CH, D), jnp.float32)
    zeros128 = jnp.zeros((RPT, D), jnp.float32)

    deg = _sc_degree(idx2, zeros128, ones128)
    deg_out = deg[:N]
    deg_in = deg[N_PAD:N_PAD + N]

    hn = _tc_first(features, W0, deg_out)
    part = _sc_scatter(hn, src_m, dst_m, zeros128)
    p0, p1 = part[:N], part[N_PAD:N_PAD + N]
    hn = _tc_mid(p0, p1, deg_in, deg_out, b0.reshape(1, D), W1)
    part = _sc_scatter(hn, src_m, dst_m, zeros128)
    p0, p1 = part[:N], part[N_PAD:N_PAD + N]
    hn = _tc_mid(p0, p1, deg_in, deg_out, b1.reshape(1, D), W2)
    part = _sc_scatter(hn, src_m, dst_m, zeros128)
    p0, p1 = part[:N], part[N_PAD:N_PAD + N]

    w = jax.nn.softmax(pool_weight, axis=0)
    return _tc_final(
        p0, p1, deg_in, b2.reshape(1, D),
        w[0].reshape(1, 1), w[1].reshape(1, 1),
    )
